# Initial kernel scaffold; baseline (speedup 1.0000x reference)
import jax
import jax.numpy as jnp
from jax.experimental import pallas as pl


def kernel(inputs):
    raise NotImplementedError("write your pallas kernel here")


# speedup vs baseline: 41.9389x; 41.9389x over previous
import functools

import jax
import jax.numpy as jnp
from jax import lax
from jax.experimental import pallas as pl
from jax.experimental.pallas import tpu as pltpu
from jax.experimental.pallas import tpu_sc as plsc

B = 16
NUM_FRAMES = 998
FRAME_SIZE = 400
FRAME_SHIFT = 160
SAMPLES = 160000
NW = 32
FPW = (B * NUM_FRAMES) // NW
SPAN = FRAME_SHIFT * (FPW - 1) + FRAME_SIZE


@functools.partial(
    pl.kernel,
    out_type=jax.ShapeDtypeStruct((B * NUM_FRAMES * FRAME_SIZE,), jnp.float32),
    mesh=plsc.VectorSubcoreMesh(core_axis_name="c", subcore_axis_name="s"),
    scratch_types=[
        pltpu.VMEM((SPAN,), jnp.float32),
        pltpu.SemaphoreType.DMA,
    ],
)
def _frame_copy(in_hbm, out_hbm, in_v, sem):
    wid = lax.axis_index("s") * 2 + lax.axis_index("c")
    b = wid // 2
    n0 = (wid % 2) * FPW
    in_base = b * SAMPLES + n0 * FRAME_SHIFT
    out_base = (b * NUM_FRAMES + n0) * FRAME_SIZE

    pltpu.sync_copy(in_hbm.at[pl.ds(in_base, SPAN)], in_v)

    def issue(t, carry):
        pltpu.async_copy(
            in_v.at[pl.ds(t * FRAME_SHIFT, FRAME_SIZE)],
            out_hbm.at[pl.ds(out_base + t * FRAME_SIZE, FRAME_SIZE)],
            sem,
        )
        return carry

    lax.fori_loop(0, FPW, issue, 0)

    def drain(words):
        pltpu.make_async_copy(
            in_hbm.at[pl.ds(0, words)], in_v.at[pl.ds(0, words)], sem
        ).wait()

    total = FPW * FRAME_SIZE
    drain(SPAN)
    drain(SPAN)
    drain(total - 2 * SPAN)


def kernel(inputs):
    out = _frame_copy(inputs.reshape(B * SAMPLES))
    return out.reshape(B, NUM_FRAMES, FRAME_SIZE)
